# Initial kernel scaffold; baseline (speedup 1.0000x reference)
import jax
import jax.numpy as jnp
from jax.experimental import pallas as pl


def kernel(inputs, weights, bias):
    raise NotImplementedError("write your pallas kernel here")


# speedup vs baseline: 3.4502x; 3.4502x over previous
import functools

import jax
import jax.numpy as jnp
from jax.experimental import pallas as pl
from jax.experimental.pallas import tpu as pltpu


def _round_up(x, m):
    return (x + m - 1) // m * m


def _conv_body(x_ref, w_ref, b_ref, o_ref, *, W, HW, KH, KW, pad,
               OWv, M_valid, FRONT, L):
    xb = x_ref[...].astype(jnp.bfloat16)
    xp = jnp.pad(xb, ((0, 0), (FRONT, L - FRONT - HW)))
    lane = jax.lax.broadcasted_iota(jnp.int32, (1, HW), 1)
    col = lane % W
    pieces = []
    for kh in range(KH):
        for kw in range(KW):
            dh, dw = kh - pad, kw - pad
            off = FRONT + dh * W + dw
            s = jax.lax.slice_in_dim(xp, off, off + HW, axis=1)
            if dw < 0:
                s = jnp.where(col >= -dw, s, jnp.bfloat16(0))
            elif dw > 0:
                s = jnp.where(col < W - dw, s, jnp.bfloat16(0))
            pieces.append(s)
    xs = jnp.concatenate(pieces, axis=0)
    acc = jnp.dot(w_ref[...], xs, preferred_element_type=jnp.float32)
    valid = (col < OWv) & (lane < M_valid)
    o_ref[...] = jnp.where(valid, acc + b_ref[...], jnp.float32(0))


def kernel(inputs, weights, bias):
    B, C, H, W = inputs.shape
    O, Cw, KH, KW = weights.shape
    assert C == Cw, "channel mismatch"
    pad = 1
    OHv = H - KH + 1
    OWv = W - KW + 1
    HW = H * W
    FRONT = _round_up(pad * W + pad, 128)
    L = _round_up(FRONT + HW + pad * W + pad, 128)

    x_flat = inputs.reshape(B, C, HW)
    w_k = (weights.astype(jnp.float32).transpose(0, 2, 3, 1)
           .reshape(O, KH * KW * C).astype(jnp.bfloat16))
    b_k = jnp.reshape(bias, (-1,)).astype(jnp.float32).reshape(O, 1)

    body = functools.partial(
        _conv_body, W=W, HW=HW, KH=KH, KW=KW, pad=pad,
        OWv=OWv, M_valid=OHv * W, FRONT=FRONT, L=L)

    out_flat = pl.pallas_call(
        body,
        out_shape=jax.ShapeDtypeStruct((B, O, HW), jnp.float32),
        grid=(B,),
        in_specs=[
            pl.BlockSpec((None, C, HW), lambda b: (b, 0, 0)),
            pl.BlockSpec((O, KH * KW * C), lambda b: (0, 0)),
            pl.BlockSpec((O, 1), lambda b: (0, 0)),
        ],
        out_specs=pl.BlockSpec((None, O, HW), lambda b: (b, 0, 0)),
        compiler_params=pltpu.CompilerParams(
            dimension_semantics=("parallel",),
            vmem_limit_bytes=int(48 * 1024 * 1024),
        ),
        cost_estimate=pl.CostEstimate(
            flops=2 * B * HW * KH * KW * C * O,
            transcendentals=0,
            bytes_accessed=int(4 * B * C * HW + 2 * O * KH * KW * C
                               + 4 * B * O * HW),
        ),
    )(x_flat, w_k, b_k)

    return out_flat.reshape(B, O, H, W)
